# Initial kernel scaffold; baseline (speedup 1.0000x reference)
import jax
import jax.numpy as jnp
from jax.experimental import pallas as pl


def kernel(batch, donor, assay, W_batch, W_donor, W_assay):
    raise NotImplementedError("write your pallas kernel here")


# speedup vs baseline: 1.5260x; 1.5260x over previous
import functools

import jax
import jax.numpy as jnp
from jax import lax
from jax.experimental import pallas as pl
from jax.experimental.pallas import tpu as pltpu
from jax.experimental.pallas import tpu_sc as plsc

B = 16384
D_BATCH, D_DONOR, D_ASSAY = 64, 64, 32
D_OUT = D_BATCH + D_DONOR + D_ASSAY

NC, NS = 2, 16
NW = NC * NS
BPW = B // NW
CHUNK = 128
NCH = BPW // CHUNK

_mesh = plsc.VectorSubcoreMesh(core_axis_name="c", subcore_axis_name="s")


@functools.partial(
    pl.kernel,
    out_type=jax.ShapeDtypeStruct((B, D_OUT), jnp.float32),
    mesh=_mesh,
    scratch_types=[
        pltpu.VMEM((NCH, CHUNK), jnp.int32),
        pltpu.VMEM((NCH, CHUNK), jnp.int32),
        pltpu.VMEM((NCH, CHUNK), jnp.int32),
        pltpu.VMEM((BPW, D_BATCH), jnp.float32),
        pltpu.VMEM((BPW, D_DONOR), jnp.float32),
        pltpu.VMEM((BPW, D_ASSAY), jnp.float32),
        pltpu.SemaphoreType.DMA,
    ],
    compiler_params=pltpu.CompilerParams(use_tc_tiling_on_sc=False),
)
def _embed_concat(b_idx, d_idx, a_idx, Wb, Wd, Wa, out,
                  ib, idn, ia, rb, rd, ra, sem):
    wid = lax.axis_index("c") * NS + lax.axis_index("s")
    base = wid * BPW
    pltpu.sync_copy(b_idx.at[pl.ds(wid * NCH, NCH)], ib)
    pltpu.sync_copy(d_idx.at[pl.ds(wid * NCH, NCH)], idn)
    pltpu.sync_copy(a_idx.at[pl.ds(wid * NCH, NCH)], ia)
    copies = []
    for j in range(NCH):
        sl = pl.ds(j * CHUNK, CHUNK)
        copies.append(pltpu.async_copy(Wb.at[ib.at[j]], rb.at[sl], sem))
        copies.append(pltpu.async_copy(Wd.at[idn.at[j]], rd.at[sl], sem))
        copies.append(pltpu.async_copy(Wa.at[ia.at[j]], ra.at[sl], sem))
    for c in copies:
        c.wait()
    out_rows = pl.ds(base, BPW)
    pltpu.sync_copy(rb, out.at[out_rows, pl.ds(0, D_BATCH)])
    pltpu.sync_copy(rd, out.at[out_rows, pl.ds(D_BATCH, D_DONOR)])
    pltpu.sync_copy(ra, out.at[out_rows, pl.ds(D_BATCH + D_DONOR, D_ASSAY)])


def kernel(batch, donor, assay, W_batch, W_donor, W_assay):
    b2 = batch.astype(jnp.int32).reshape(NW * NCH, CHUNK)
    d2 = donor.astype(jnp.int32).reshape(NW * NCH, CHUNK)
    a2 = assay.astype(jnp.int32).reshape(NW * NCH, CHUNK)
    return _embed_concat(b2, d2, a2, W_batch, W_donor, W_assay)
